# Initial kernel scaffold; baseline (speedup 1.0000x reference)
import functools

import jax
import jax.numpy as jnp
from jax import lax
from jax.experimental import pallas as pl
from jax.experimental.pallas import tpu as pltpu
from jax.experimental.pallas import tpu_sc as plsc

B = 16384
D = 64
BINS = 10000
NF = 6
NC = 2
NS = 16
NW = NC * NS
RPW = B // NW
LANES = 16


def _tower_body(eid_hbm, table_hbm, feats_hbm, w_hbm, b_hbm, out_hbm,
                idx_v, rows_v, feats_v, w_v, b_v, dense_v, sem):
    wid = lax.axis_index("s") * NC + lax.axis_index("c")
    base = wid * RPW

    pltpu.sync_copy(eid_hbm.at[pl.ds(base, RPW)], idx_v)

    def _mod_body(i, _):
        v = idx_v[pl.ds(i * LANES, LANES)]
        idx_v[pl.ds(i * LANES, LANES)] = lax.rem(v, BINS)
        return 0

    lax.fori_loop(0, RPW // LANES, _mod_body, 0)

    gather = pltpu.async_copy(table_hbm.at[idx_v], rows_v, sem)

    pltpu.sync_copy(w_hbm, w_v)
    pltpu.sync_copy(b_hbm, b_v)
    for f in range(NF):
        pltpu.sync_copy(feats_hbm.at[f, pl.ds(base, RPW)], feats_v.at[f])

    gather.wait()
    emb_out = pltpu.async_copy(
        rows_v, out_hbm.at[pl.ds(base, RPW), pl.ds(0, D)], sem)

    for f in range(NF):
        wf = [w_v[f, pl.ds(c * LANES, LANES)] for c in range(D // LANES)]
        bf = [b_v[f, pl.ds(c * LANES, LANES)] for c in range(D // LANES)]

        def _chunk_body(ii, _, f=f, wf=wf, bf=bf):
            fch = feats_v[f, pl.ds(ii * LANES, LANES)]
            for lane in range(LANES):
                fb = lax.broadcast_in_dim(
                    lax.slice(fch, (lane,), (lane + 1,)), (LANES,), (0,))
                r = ii * LANES + lane
                for c in range(D // LANES):
                    dense_v[r, pl.ds(c * LANES, LANES)] = fb * wf[c] + bf[c]
            return 0

        lax.fori_loop(0, RPW // LANES, _chunk_body, 0)
        pltpu.sync_copy(
            dense_v, out_hbm.at[pl.ds(base, RPW), pl.ds((f + 1) * D, D)])

    emb_out.wait()


@functools.partial(
    pl.kernel,
    mesh=plsc.VectorSubcoreMesh(core_axis_name="c", subcore_axis_name="s"),
    out_type=jax.ShapeDtypeStruct((B, (NF + 1) * D), jnp.float32),
    scratch_types=[
        pltpu.VMEM((RPW,), jnp.int32),
        pltpu.VMEM((RPW, D), jnp.float32),
        pltpu.VMEM((NF, RPW), jnp.float32),
        pltpu.VMEM((NF, D), jnp.float32),
        pltpu.VMEM((NF, D), jnp.float32),
        pltpu.VMEM((RPW, D), jnp.float32),
        pltpu.SemaphoreType.DMA,
    ],
)
def _tower_kernel(eid, table, feats, w, b, out,
                  idx_v, rows_v, feats_v, w_v, b_v, dense_v, sem):
    _tower_body(eid, table, feats, w, b, out,
                idx_v, rows_v, feats_v, w_v, b_v, dense_v, sem)


def kernel(engagement_id, table,
           feat_type, W_type, b_type,
           feat_duration, W_duration, b_duration,
           feat_difficulty, W_difficulty, b_difficulty,
           feat_prerequisites, W_prerequisites, b_prerequisites,
           feat_popularity, W_popularity, b_popularity,
           feat_success_rate, W_success_rate, b_success_rate):
    feats = jnp.stack([feat_type, feat_duration, feat_difficulty,
                       feat_prerequisites, feat_popularity, feat_success_rate])
    w = jnp.concatenate([W_type, W_duration, W_difficulty,
                         W_prerequisites, W_popularity, W_success_rate], axis=0)
    bias = jnp.stack([b_type, b_duration, b_difficulty,
                      b_prerequisites, b_popularity, b_success_rate])
    return _tower_kernel(engagement_id, table, feats, w, bias)


# speedup vs baseline: 1.3303x; 1.3303x over previous
import functools

import jax
import jax.numpy as jnp
from jax import lax
from jax.experimental import pallas as pl
from jax.experimental.pallas import tpu as pltpu
from jax.experimental.pallas import tpu_sc as plsc

B = 16384
D = 64
OUTW = 7 * D
DENSEW = 6 * D
BINS = 10000
NF = 6
NC = 2
NS = 16
NW = NC * NS
RPW = B // NW
SB = 64
NSB = RPW // SB
L = 16
CH = D // L


def _tower_body(eid_hbm, table_hbm, feat_refs, w_refs, b_refs, out_hbm,
                idx_v, rows_v, feats_v, w_v, b_v, blk_v,
                gsem, esem, dsem0, dsem1):
    wid = lax.axis_index("s") * NC + lax.axis_index("c")
    base = wid * RPW

    pltpu.sync_copy(eid_hbm.at[pl.ds(base, RPW)], idx_v)

    def _mod_body(i, _):
        v = idx_v[pl.ds(i * L, L)]
        idx_v[pl.ds(i * L, L)] = lax.rem(v, BINS)
        return 0

    lax.fori_loop(0, RPW // L, _mod_body, 0)

    gather = pltpu.async_copy(table_hbm.at[idx_v], rows_v, gsem)

    for f in range(NF):
        pltpu.sync_copy(w_refs[f].at[0], w_v.at[pl.ds(f * D, D)])
        pltpu.sync_copy(b_refs[f], b_v.at[pl.ds(f * D, D)])
        pltpu.sync_copy(feat_refs[f].at[pl.ds(base, RPW)],
                        feats_v.at[pl.ds(f * RPW, RPW)])

    gather.wait()
    emb = pltpu.async_copy(
        rows_v, out_hbm.at[pl.ds(base, RPW), pl.ds(0, D)], esem)

    dsems = (dsem0, dsem1)

    def _dense_block(s, buf, dsem):
        rowbase = s * SB
        blk = blk_v.at[buf]

        def _group(g, _):
            rg = g * L
            for f in range(NF):
                wf = [w_v[pl.ds(f * D + c * L, L)] for c in range(CH)]
                bf = [b_v[pl.ds(f * D + c * L, L)] for c in range(CH)]
                fch = feats_v[pl.ds(f * RPW + rowbase + rg, L)]
                for lane in range(L):
                    fb = lax.gather(
                        fch, jnp.full((L, 1), lane, jnp.int32),
                        lax.GatherDimensionNumbers(
                            offset_dims=(), collapsed_slice_dims=(0,),
                            start_index_map=(0,)),
                        slice_sizes=(1,),
                        mode=lax.GatherScatterMode.PROMISE_IN_BOUNDS)
                    r = rg + lane
                    for c in range(CH):
                        blk[r, pl.ds(f * D + c * L, L)] = fb * wf[c] + bf[c]
            return 0

        lax.fori_loop(0, SB // L, _group, 0)
        pltpu.async_copy(
            blk, out_hbm.at[pl.ds(base + rowbase, SB), pl.ds(D, DENSEW)],
            dsem)

    def _wait_dense(buf, dsem):
        pltpu.make_async_copy(
            blk_v.at[buf],
            out_hbm.at[pl.ds(0, SB), pl.ds(D, DENSEW)],
            dsem).wait()

    def _pair(p, _):
        s = p * 2
        pl.when(p > 0)(lambda: _wait_dense(0, dsems[0]))
        _dense_block(s, 0, dsems[0])
        pl.when(p > 0)(lambda: _wait_dense(1, dsems[1]))
        _dense_block(s + 1, 1, dsems[1])
        return 0

    lax.fori_loop(0, NSB // 2, _pair, 0)

    _wait_dense(0, dsems[0])
    _wait_dense(1, dsems[1])
    emb.wait()


@functools.partial(
    pl.kernel,
    mesh=plsc.VectorSubcoreMesh(core_axis_name="c", subcore_axis_name="s"),
    out_type=jax.ShapeDtypeStruct((B, OUTW), jnp.float32),
    compiler_params=pltpu.CompilerParams(use_tc_tiling_on_sc=False),
    scratch_types=[
        pltpu.VMEM((RPW,), jnp.int32),
        pltpu.VMEM((RPW, D), jnp.float32),
        pltpu.VMEM((NF * RPW,), jnp.float32),
        pltpu.VMEM((NF * D,), jnp.float32),
        pltpu.VMEM((NF * D,), jnp.float32),
        pltpu.VMEM((2, SB, DENSEW), jnp.float32),
        pltpu.SemaphoreType.DMA,
        pltpu.SemaphoreType.DMA,
        pltpu.SemaphoreType.DMA,
        pltpu.SemaphoreType.DMA,
    ],
)
def _tower_kernel(eid, table,
                  f0, w0, b0, f1, w1, b1, f2, w2, b2,
                  f3, w3, b3, f4, w4, b4, f5, w5, b5,
                  out,
                  idx_v, rows_v, feats_v, w_v, b_v, blk_v,
                  gsem, esem, dsem0, dsem1):
    _tower_body(eid, table,
                (f0, f1, f2, f3, f4, f5),
                (w0, w1, w2, w3, w4, w5),
                (b0, b1, b2, b3, b4, b5),
                out,
                idx_v, rows_v, feats_v, w_v, b_v, blk_v,
                gsem, esem, dsem0, dsem1)


def kernel(engagement_id, table,
           feat_type, W_type, b_type,
           feat_duration, W_duration, b_duration,
           feat_difficulty, W_difficulty, b_difficulty,
           feat_prerequisites, W_prerequisites, b_prerequisites,
           feat_popularity, W_popularity, b_popularity,
           feat_success_rate, W_success_rate, b_success_rate):
    return _tower_kernel(
        engagement_id, table,
        feat_type, W_type, b_type,
        feat_duration, W_duration, b_duration,
        feat_difficulty, W_difficulty, b_difficulty,
        feat_prerequisites, W_prerequisites, b_prerequisites,
        feat_popularity, W_popularity, b_popularity,
        feat_success_rate, W_success_rate, b_success_rate)


# speedup vs baseline: 1.9741x; 1.4839x over previous
import functools

import jax
import jax.numpy as jnp
from jax import lax
from jax.experimental import pallas as pl
from jax.experimental.pallas import tpu as pltpu
from jax.experimental.pallas import tpu_sc as plsc

B = 16384
D = 64
OUTW = 7 * D
BINS = 10000
NF = 6
NC = 2
NS = 16
NW = NC * NS
RPW = B // NW
RT = RPW // 128
NTC = OUTW // 8
L = 16
TILE = 8 * 128
BAND = RT * TILE


def _tower_body(eid_hbm, table_hbm, feat_refs, w_refs, b_refs, out_hbm,
                idx_v, rows_v, feats_v, w_v, b_v, band_v, pair0_v, pair1_v,
                gsem, bsem0, bsem1, psem0, psem1):
    wid = lax.axis_index("s") * NC + lax.axis_index("c")
    base = wid * RPW
    j0 = wid * RT

    pltpu.sync_copy(eid_hbm.at[pl.ds(base, RPW)], idx_v)

    def _mod_body(i, _):
        v = idx_v[pl.ds(i * L, L)]
        idx_v[pl.ds(i * L, L)] = lax.rem(v, BINS)
        return 0

    lax.fori_loop(0, RPW // L, _mod_body, 0)

    gather = pltpu.async_copy(table_hbm.at[idx_v], rows_v, gsem)

    for f in range(NF):
        pltpu.sync_copy(w_refs[f].at[0], w_v.at[pl.ds(f * D, D)])
        pltpu.sync_copy(b_refs[f], b_v.at[pl.ds(f * D, D)])
        pltpu.sync_copy(feat_refs[f].at[pl.ds(base, RPW)],
                        feats_v.at[pl.ds(f * RPW, RPW)])

    bsems = (bsem0, bsem1)

    def _band_wait(buf, sem):
        pltpu.make_async_copy(
            band_v.at[buf], out_hbm.at[pl.ds(0, BAND)], sem).wait()

    def _dense_band(i, half, buf, sem):
        band = band_v.at[buf]
        k = (i - 8) * 8
        f = k // D
        c_in_f = k - f * D
        ch16 = c_in_f - 8 * half
        wch = w_v[pl.ds(f * D + ch16, L)]
        bch = b_v[pl.ds(f * D + ch16, L)]
        wb = []
        bb = []
        for c in range(8):
            lane = jnp.full((L, 1), half * 8 + c, jnp.int32)
            dn = lax.GatherDimensionNumbers(
                offset_dims=(), collapsed_slice_dims=(0,),
                start_index_map=(0,))
            wb.append(lax.gather(wch, lane, dn, slice_sizes=(1,),
                                 mode=lax.GatherScatterMode.PROMISE_IN_BOUNDS))
            bb.append(lax.gather(bch, lane, dn, slice_sizes=(1,),
                                 mode=lax.GatherScatterMode.PROMISE_IN_BOUNDS))
        for jj in range(RT):
            fch = [feats_v[pl.ds(f * RPW + jj * 128 + h * L, L)]
                   for h in range(8)]
            for c in range(8):
                for h in range(8):
                    band[pl.ds(jj * TILE + c * 128 + h * L, L)] = (
                        fch[h] * wb[c] + bb[c])
        pltpu.async_copy(
            band, out_hbm.at[pl.ds((i * 128 + j0) * TILE, BAND)], sem)

    def _dense_pair(p, _):
        i = 8 + 2 * p
        pl.when(p > 0)(lambda: _band_wait(0, bsems[0]))
        _dense_band(i, 0, 0, bsems[0])
        pl.when(p > 0)(lambda: _band_wait(1, bsems[1]))
        _dense_band(i + 1, 1, 1, bsems[1])
        return 0

    lax.fori_loop(0, (NTC - 8) // 2, _dense_pair, 0)

    gather.wait()
    psems = (psem0, psem1)
    lanes = lax.iota(jnp.int32, L)
    pattern = (lanes // 8) * BAND + (lanes % 8) * 128

    pairs = (pair0_v, pair1_v)
    lanes16 = lax.iota(jnp.int32, L)

    def _pair_wait(buf, sem):
        pltpu.make_async_copy(
            pairs[buf], out_hbm.at[pl.ds(0, 2 * BAND)], sem).wait()

    def _emb_pair(p, buf, sem):
        pair = pairs[buf]

        def _rows(q, _):
            rr = q * L
            jj = rr // 128
            sbase = jj * TILE + (rr - jj * 128)
            ridx = lanes16 + rr
            for c in range(L):
                cg = p * L + c
                v = plsc.load_gather(
                    rows_v, [ridx, jnp.full((L,), cg, jnp.int32)])
                pair[pl.ds((c // 8) * BAND + (c % 8) * 128 + sbase, L)] = v
            return 0

        lax.fori_loop(0, RPW // L, _rows, 0)
        pltpu.async_copy(
            pair.at[pl.ds(0, BAND)],
            out_hbm.at[pl.ds((2 * p * 128 + j0) * TILE, BAND)], sem)
        pltpu.async_copy(
            pair.at[pl.ds(BAND, BAND)],
            out_hbm.at[pl.ds(((2 * p + 1) * 128 + j0) * TILE, BAND)], sem)

    for p in range(4):
        if p >= 2:
            _pair_wait(p % 2, psems[p % 2])
        _emb_pair(p, p % 2, psems[p % 2])

    _band_wait(0, bsems[0])
    _band_wait(1, bsems[1])
    _pair_wait(0, psems[0])
    _pair_wait(1, psems[1])


@functools.partial(
    pl.kernel,
    mesh=plsc.VectorSubcoreMesh(core_axis_name="c", subcore_axis_name="s"),
    out_type=jax.ShapeDtypeStruct((B * OUTW,), jnp.float32),
    compiler_params=pltpu.CompilerParams(use_tc_tiling_on_sc=False,
                                         needs_layout_passes=False),
    scratch_types=[
        pltpu.VMEM((RPW,), jnp.int32),
        pltpu.VMEM((RPW, D), jnp.float32),
        pltpu.VMEM((NF * RPW,), jnp.float32),
        pltpu.VMEM((NF * D,), jnp.float32),
        pltpu.VMEM((NF * D,), jnp.float32),
        pltpu.VMEM((2, BAND), jnp.float32),
        pltpu.VMEM((2 * BAND,), jnp.float32),
        pltpu.VMEM((2 * BAND,), jnp.float32),
        pltpu.SemaphoreType.DMA,
        pltpu.SemaphoreType.DMA,
        pltpu.SemaphoreType.DMA,
        pltpu.SemaphoreType.DMA,
        pltpu.SemaphoreType.DMA,
    ],
)
def _tower_kernel(eid, table,
                  f0, w0, b0, f1, w1, b1, f2, w2, b2,
                  f3, w3, b3, f4, w4, b4, f5, w5, b5,
                  out,
                  idx_v, rows_v, feats_v, w_v, b_v, band_v, pair0_v, pair1_v,
                  gsem, bsem0, bsem1, psem0, psem1):
    _tower_body(eid, table,
                (f0, f1, f2, f3, f4, f5),
                (w0, w1, w2, w3, w4, w5),
                (b0, b1, b2, b3, b4, b5),
                out,
                idx_v, rows_v, feats_v, w_v, b_v, band_v, pair0_v, pair1_v,
                gsem, bsem0, bsem1, psem0, psem1)


def kernel(engagement_id, table,
           feat_type, W_type, b_type,
           feat_duration, W_duration, b_duration,
           feat_difficulty, W_difficulty, b_difficulty,
           feat_prerequisites, W_prerequisites, b_prerequisites,
           feat_popularity, W_popularity, b_popularity,
           feat_success_rate, W_success_rate, b_success_rate):
    flat = _tower_kernel(
        engagement_id, table,
        feat_type, W_type, b_type,
        feat_duration, W_duration, b_duration,
        feat_difficulty, W_difficulty, b_difficulty,
        feat_prerequisites, W_prerequisites, b_prerequisites,
        feat_popularity, W_popularity, b_popularity,
        feat_success_rate, W_success_rate, b_success_rate)
    return (flat.reshape(NTC, B // 128, 8, 128)
            .transpose(1, 3, 0, 2)
            .reshape(B, OUTW))


# speedup vs baseline: 2.2159x; 1.1225x over previous
import functools

import jax
import jax.numpy as jnp
from jax import lax
from jax.experimental import pallas as pl
from jax.experimental.pallas import tpu as pltpu
from jax.experimental.pallas import tpu_sc as plsc

B = 16384
D = 64
OUTW = 7 * D
BINS = 10000
NF = 6
NC = 2
NS = 16
NW = NC * NS
RPW = B // NW
RT = RPW // 128
NTC = OUTW // 8
L = 16
TILE = 8 * 128
BAND = RT * TILE


def _tower_body(eid_hbm, table_hbm, feat_refs, w_refs, b_refs, out_hbm,
                idx_v, rows_v, feats_v, w_v, b_v, band_v, pair0_v, pair1_v,
                t16_v, gsem, bsem0, bsem1, psem0, psem1):
    wid = lax.axis_index("s") * NC + lax.axis_index("c")
    base = wid * RPW
    j0 = wid * RT

    pltpu.sync_copy(eid_hbm.at[pl.ds(base, RPW)], idx_v)

    def _mod_body(i, _):
        v = idx_v[pl.ds(i * L, L)]
        idx_v[pl.ds(i * L, L)] = lax.rem(v, BINS)
        return 0

    lax.fori_loop(0, RPW // L, _mod_body, 0)

    gather = pltpu.async_copy(table_hbm.at[idx_v], rows_v, gsem)

    for f in range(NF):
        pltpu.sync_copy(w_refs[f].at[0], w_v.at[pl.ds(f * D, D)])
        pltpu.sync_copy(b_refs[f], b_v.at[pl.ds(f * D, D)])
        pltpu.sync_copy(feat_refs[f].at[pl.ds(base, RPW)],
                        feats_v.at[pl.ds(f * RPW, RPW)])

    bsems = (bsem0, bsem1)

    def _band_wait(buf, sem):
        pltpu.make_async_copy(
            band_v.at[buf], out_hbm.at[pl.ds(0, BAND)], sem).wait()

    def _dense_band(i, half, buf, sem):
        band = band_v.at[buf]
        k = (i - 8) * 8
        f = k // D
        c_in_f = k - f * D
        ch16 = c_in_f - 8 * half
        wch = w_v[pl.ds(f * D + ch16, L)]
        bch = b_v[pl.ds(f * D + ch16, L)]
        wb = []
        bb = []
        for c in range(8):
            lane = jnp.full((L, 1), half * 8 + c, jnp.int32)
            dn = lax.GatherDimensionNumbers(
                offset_dims=(), collapsed_slice_dims=(0,),
                start_index_map=(0,))
            wb.append(lax.gather(wch, lane, dn, slice_sizes=(1,),
                                 mode=lax.GatherScatterMode.PROMISE_IN_BOUNDS))
            bb.append(lax.gather(bch, lane, dn, slice_sizes=(1,),
                                 mode=lax.GatherScatterMode.PROMISE_IN_BOUNDS))
        for jj in range(RT):
            fch = [feats_v[pl.ds(f * RPW + jj * 128 + h * L, L)]
                   for h in range(8)]
            for c in range(8):
                for h in range(8):
                    band[pl.ds(jj * TILE + c * 128 + h * L, L)] = (
                        fch[h] * wb[c] + bb[c])
        pltpu.async_copy(
            band, out_hbm.at[pl.ds((i * 128 + j0) * TILE, BAND)], sem)

    def _dense_pair(p, _):
        i = 8 + 2 * p
        pl.when(p > 0)(lambda: _band_wait(0, bsems[0]))
        _dense_band(i, 0, 0, bsems[0])
        pl.when(p > 0)(lambda: _band_wait(1, bsems[1]))
        _dense_band(i + 1, 1, 1, bsems[1])
        return 0

    lax.fori_loop(0, (NTC - 8) // 2, _dense_pair, 0)

    gather.wait()
    psems = (psem0, psem1)
    pairs = (pair0_v, pair1_v)
    lanes16 = lax.iota(jnp.int32, L)
    colpat = [(lanes16 + k) % L for k in range(L)]
    qpat = [18 * ((c - lanes16) % L) + lanes16 for c in range(L)]

    def _pair_wait(buf, sem):
        pltpu.make_async_copy(
            pairs[buf], out_hbm.at[pl.ds(0, 2 * BAND)], sem).wait()

    def _emb_pair(p, buf, sem):
        pair = pairs[buf]
        cidx = [cp + p * L for cp in colpat]

        def _rows(q, _):
            rr = q * L
            jj = rr // 128
            sbase = jj * TILE + (rr - jj * 128)
            ridx = lanes16 + rr
            for k in range(L):
                t16_v[pl.ds(k * 18, L)] = plsc.load_gather(
                    rows_v, [ridx, cidx[k]])
            for c in range(L):
                v = plsc.load_gather(t16_v, [qpat[c]])
                pair[pl.ds((c // 8) * BAND + (c % 8) * 128 + sbase, L)] = v
            return 0

        lax.fori_loop(0, RPW // L, _rows, 0)
        pltpu.async_copy(
            pair.at[pl.ds(0, BAND)],
            out_hbm.at[pl.ds((2 * p * 128 + j0) * TILE, BAND)], sem)
        pltpu.async_copy(
            pair.at[pl.ds(BAND, BAND)],
            out_hbm.at[pl.ds(((2 * p + 1) * 128 + j0) * TILE, BAND)], sem)

    for p in range(4):
        if p >= 2:
            _pair_wait(p % 2, psems[p % 2])
        _emb_pair(p, p % 2, psems[p % 2])

    _band_wait(0, bsems[0])
    _band_wait(1, bsems[1])
    _pair_wait(0, psems[0])
    _pair_wait(1, psems[1])


@functools.partial(
    pl.kernel,
    mesh=plsc.VectorSubcoreMesh(core_axis_name="c", subcore_axis_name="s"),
    out_type=jax.ShapeDtypeStruct((B * OUTW,), jnp.float32),
    compiler_params=pltpu.CompilerParams(use_tc_tiling_on_sc=False,
                                         needs_layout_passes=False),
    scratch_types=[
        pltpu.VMEM((RPW,), jnp.int32),
        pltpu.VMEM((RPW, D), jnp.float32),
        pltpu.VMEM((NF * RPW,), jnp.float32),
        pltpu.VMEM((NF * D,), jnp.float32),
        pltpu.VMEM((NF * D,), jnp.float32),
        pltpu.VMEM((2, BAND), jnp.float32),
        pltpu.VMEM((2 * BAND,), jnp.float32),
        pltpu.VMEM((2 * BAND,), jnp.float32),
        pltpu.VMEM((16 * 18,), jnp.float32),
        pltpu.SemaphoreType.DMA,
        pltpu.SemaphoreType.DMA,
        pltpu.SemaphoreType.DMA,
        pltpu.SemaphoreType.DMA,
        pltpu.SemaphoreType.DMA,
    ],
)
def _tower_kernel(eid, table,
                  f0, w0, b0, f1, w1, b1, f2, w2, b2,
                  f3, w3, b3, f4, w4, b4, f5, w5, b5,
                  out,
                  idx_v, rows_v, feats_v, w_v, b_v, band_v, pair0_v, pair1_v,
                  t16_v, gsem, bsem0, bsem1, psem0, psem1):
    _tower_body(eid, table,
                (f0, f1, f2, f3, f4, f5),
                (w0, w1, w2, w3, w4, w5),
                (b0, b1, b2, b3, b4, b5),
                out,
                idx_v, rows_v, feats_v, w_v, b_v, band_v, pair0_v, pair1_v,
                t16_v, gsem, bsem0, bsem1, psem0, psem1)


def kernel(engagement_id, table,
           feat_type, W_type, b_type,
           feat_duration, W_duration, b_duration,
           feat_difficulty, W_difficulty, b_difficulty,
           feat_prerequisites, W_prerequisites, b_prerequisites,
           feat_popularity, W_popularity, b_popularity,
           feat_success_rate, W_success_rate, b_success_rate):
    flat = _tower_kernel(
        engagement_id, table,
        feat_type, W_type, b_type,
        feat_duration, W_duration, b_duration,
        feat_difficulty, W_difficulty, b_difficulty,
        feat_prerequisites, W_prerequisites, b_prerequisites,
        feat_popularity, W_popularity, b_popularity,
        feat_success_rate, W_success_rate, b_success_rate)
    return (flat.reshape(NTC, B // 128, 8, 128)
            .transpose(1, 3, 0, 2)
            .reshape(B, OUTW))
